# baseline (speedup 1.0000x reference)
import functools

import jax
import jax.numpy as jnp
from jax import lax
from jax.experimental import pallas as pl
from jax.experimental.pallas import tpu as pltpu
from jax.experimental.pallas import tpu_sc as plsc

B = 64
N = 262144
C = 2

_NC = 2
_NS = 16
_NW = _NC * _NS
_RPW = N // _NW
_WPW = _RPW * C


_mesh = plsc.VectorSubcoreMesh(core_axis_name="c", subcore_axis_name="s")


@functools.partial(
    pl.kernel,
    mesh=_mesh,
    compiler_params=pltpu.CompilerParams(
        needs_layout_passes=False, use_tc_tiling_on_sc=False
    ),
    out_type=jax.ShapeDtypeStruct((B, N * C), jnp.float32),
    scratch_types=[
        pltpu.VMEM((_RPW,), jnp.int32),
        pltpu.VMEM((_WPW,), jnp.int32),
        pltpu.VMEM((_WPW,), jnp.float32),
        pltpu.SemaphoreType.DMA,
    ],
)
def _gather_kernel(x_hbm, perm_hbm, out_hbm, idx_v, idx2_v, words_v, sem):
    wid = lax.axis_index("s") * _NC + lax.axis_index("c")
    base = wid * _RPW
    pltpu.sync_copy(perm_hbm.at[pl.ds(base, _RPW)], idx_v)

    lanes = lax.iota(jnp.int32, 16)

    def expand(k, carry):
        p = idx_v[pl.ds(k * 16, 16)]
        w = p * 2
        pos = k * 32 + lanes * 2
        plsc.store_scatter(idx2_v, [pos], w)
        plsc.store_scatter(idx2_v, [pos + 1], w + 1)
        return carry

    lax.fori_loop(0, _RPW // 16, expand, 0)

    def body(b, carry):
        pltpu.async_copy(x_hbm.at[b].at[idx2_v], words_v, sem).wait()
        pltpu.sync_copy(words_v, out_hbm.at[b].at[pl.ds(base * C, _WPW)])
        return carry

    lax.fori_loop(0, B, body, 0)


def kernel(x, perm):
    out = _gather_kernel(x.reshape(B, N * C), perm)
    return out.reshape(B, N, C)


# speedup vs baseline: 1.0820x; 1.0820x over previous
import functools

import jax
import jax.numpy as jnp
from jax import lax
from jax.experimental import pallas as pl
from jax.experimental.pallas import tpu as pltpu
from jax.experimental.pallas import tpu_sc as plsc

B = 64
N = 262144
C = 2

_NC = 2
_NS = 16
_NW = _NC * _NS
_RPW = N // _NW
_G = 256
_NCH = _RPW // _G
_ROWW = B * C
_CW = _G * _ROWW

_mesh = plsc.VectorSubcoreMesh(core_axis_name="c", subcore_axis_name="s")
_params = pltpu.CompilerParams(needs_layout_passes=False)


@functools.partial(
    pl.kernel,
    mesh=_mesh,
    compiler_params=_params,
    out_type=jax.ShapeDtypeStruct((N, _ROWW), jnp.float32),
    scratch_types=[
        pltpu.VMEM((_CW,), jnp.float32),
        pltpu.VMEM((_G, _ROWW), jnp.float32),
        pltpu.SemaphoreType.DMA,
        pltpu.SemaphoreType.DMA,
    ],
)
def _transpose_kernel(x_hbm, xt_hbm, in_v, tile_v, lsem, wsem):
    wid = lax.axis_index("s") * _NC + lax.axis_index("c")
    lanes = lax.iota(jnp.int32, 16)
    pats = [(lanes // 2) * (2 * _G) + (lanes % 2) + 4096 * u for u in range(8)]

    def chunk(k, carry):
        j0 = wid * _RPW + k * _G
        for b in range(B):
            pltpu.async_copy(
                x_hbm.at[b].at[pl.ds(2 * j0, 2 * _G)],
                in_v.at[pl.ds(b * 2 * _G, 2 * _G)],
                lsem,
            )
        for b in range(B):
            pltpu.make_async_copy(
                x_hbm.at[b].at[pl.ds(2 * j0, 2 * _G)],
                in_v.at[pl.ds(b * 2 * _G, 2 * _G)],
                lsem,
            ).wait()

        def row(j, c2):
            t = 2 * j
            for u in range(8):
                v = plsc.load_gather(in_v, [pats[u] + t])
                tile_v[j, pl.ds(16 * u, 16)] = v
            return c2

        lax.fori_loop(0, _G, row, 0)
        pltpu.async_copy(tile_v, xt_hbm.at[pl.ds(j0, _G), :], wsem).wait()
        return carry

    lax.fori_loop(0, _NCH, chunk, 0)


@functools.partial(
    pl.kernel,
    mesh=_mesh,
    compiler_params=_params,
    out_type=jax.ShapeDtypeStruct((B, N * C), jnp.float32),
    scratch_types=[
        pltpu.VMEM((_RPW,), jnp.int32),
        pltpu.VMEM((_G, _ROWW), jnp.float32),
        pltpu.VMEM((_CW,), jnp.float32),
        pltpu.SemaphoreType.DMA,
        pltpu.SemaphoreType.DMA,
    ],
)
def _gather_kernel(xt_hbm, perm_hbm, out_hbm, idx_v, rows_v, out_v, gsem, wsem):
    wid = lax.axis_index("s") * _NC + lax.axis_index("c")
    base = wid * _RPW
    pltpu.sync_copy(perm_hbm.at[pl.ds(base, _RPW)], idx_v)

    lanes = lax.iota(jnp.int32, 16)
    lo = lanes % 2
    rowpats = [(lanes // 2) + 8 * u for u in range(8)]

    def chunk(k, carry):
        i0 = base + k * _G
        pltpu.async_copy(
            xt_hbm.at[idx_v.at[pl.ds(k * _G, _G)]], rows_v, gsem
        ).wait()

        def per_b(b, c2):
            col = lo + 2 * b

            def quad(ro, c3):
                for u in range(8):
                    r = 8 * ro + u
                    v = plsc.load_gather(rows_v, [rowpats[u] + 64 * ro, col])
                    out_v[pl.ds(2 * _G * b + 16 * r, 16)] = v
                return c3

            lax.fori_loop(0, 4, quad, 0)
            return c2

        lax.fori_loop(0, B, per_b, 0)

        for b in range(B):
            pltpu.async_copy(
                out_v.at[pl.ds(2 * _G * b, 2 * _G)],
                out_hbm.at[b].at[pl.ds(2 * i0, 2 * _G)],
                wsem,
            )
        for b in range(B):
            pltpu.make_async_copy(
                out_v.at[pl.ds(2 * _G * b, 2 * _G)],
                out_hbm.at[b].at[pl.ds(2 * i0, 2 * _G)],
                wsem,
            ).wait()
        return carry

    lax.fori_loop(0, _NCH, chunk, 0)


def kernel(x, perm):
    xt = _transpose_kernel(x.reshape(B, N * C))
    out = _gather_kernel(xt, perm)
    return out.reshape(B, N, C)


# speedup vs baseline: 2.0681x; 1.9114x over previous
import functools

import jax
import jax.numpy as jnp
from jax import lax
from jax.experimental import pallas as pl
from jax.experimental.pallas import tpu as pltpu
from jax.experimental.pallas import tpu_sc as plsc
from jax.experimental.layout import Layout, with_layout_constraint

B = 64
N = 262144
C = 2

_NC = 2
_NS = 16
_NW = _NC * _NS
_RPW = N // _NW
_TB = N // 128
_G = 256
_NCH = _RPW // _G
_ROWW = C * B
_BSTR = N * C

_mesh = plsc.VectorSubcoreMesh(core_axis_name="c", subcore_axis_name="s")
_params = pltpu.CompilerParams(needs_layout_passes=False)


def _patterns():
    lanes = lax.iota(jnp.int32, 16)
    return [lanes * 512 + 8192 * (k % 4) + 128 * (k // 4) for k in range(8)]


@functools.partial(
    pl.kernel,
    mesh=_mesh,
    compiler_params=_params,
    out_type=jax.ShapeDtypeStruct((N, _ROWW), jnp.float32),
    scratch_types=[
        pltpu.VMEM((_G * _ROWW,), jnp.float32),
        pltpu.VMEM((_G * _ROWW,), jnp.float32),
        pltpu.VMEM((_G, _ROWW), jnp.float32),
        pltpu.SemaphoreType.DMA,
        pltpu.SemaphoreType.DMA,
        pltpu.SemaphoreType.DMA,
    ],
)
def _transpose_kernel(x_hbm, xt_hbm, in0_v, in1_v, tile_v, lsem0, lsem1, wsem):
    wid = lax.axis_index("s") * _NC + lax.axis_index("c")
    pats = _patterns()
    ins = (in0_v, in1_v)
    lsems = (lsem0, lsem1)

    def src_slice(k, b):
        start = b * _BSTR + (wid * 128 + 4 * k) * 128
        return x_hbm.at[pl.ds(start, 4 * 128)]

    def loads_start(k, p):
        for b in range(B):
            pltpu.async_copy(src_slice(k, b), ins[p].at[pl.ds(b * 512, 512)], lsems[p])

    def loads_wait(k, p):
        for b in range(B):
            pltpu.make_async_copy(
                src_slice(k, b), ins[p].at[pl.ds(b * 512, 512)], lsems[p]
            ).wait()

    def compute(k, p):
        buf = ins[p]

        def half(dt):
            @plsc.parallel_loop(0, 128, unroll=2)
            def _(u):
                s = 256 * dt + u
                for kk in range(8):
                    v = plsc.load_gather(buf, [pats[kk] + s])
                    tile_v[128 * dt + u, pl.ds(16 * kk, 16)] = v

        half(0)
        half(1)

    def write(k):
        pltpu.async_copy(tile_v, xt_hbm.at[pl.ds(wid * _RPW + k * _G, _G), :], wsem)

    def write_wait(k):
        pltpu.make_async_copy(
            tile_v, xt_hbm.at[pl.ds(wid * _RPW + k * _G, _G), :], wsem
        ).wait()

    loads_start(0, 0)

    def body(m, carry):
        p = 0
        k = 2 * m
        loads_wait(k, 0)
        loads_start(k + 1, 1)
        compute(k, 0)
        write(k)
        loads_wait(k + 1, 1)

        @pl.when(m + 1 < _NCH // 2)
        def _():
            loads_start(k + 2, 0)

        write_wait(k)
        compute(k + 1, 1)
        write(k + 1)
        write_wait(k + 1)
        return carry

    lax.fori_loop(0, _NCH // 2, body, 0)


@functools.partial(
    pl.kernel,
    mesh=_mesh,
    compiler_params=_params,
    out_type=jax.ShapeDtypeStruct((B * N * C,), jnp.float32),
    scratch_types=[
        pltpu.VMEM((_RPW,), jnp.int32),
        pltpu.VMEM((_G, _ROWW), jnp.float32),
        pltpu.VMEM((_G, _ROWW), jnp.float32),
        pltpu.VMEM((_G * _ROWW,), jnp.float32),
        pltpu.SemaphoreType.DMA,
        pltpu.SemaphoreType.DMA,
        pltpu.SemaphoreType.DMA,
    ],
)
def _gather_kernel(
    xt_hbm, perm_hbm, out_hbm, idx_v, rows0_v, rows1_v, out_v, gsem0, gsem1, wsem
):
    wid = lax.axis_index("s") * _NC + lax.axis_index("c")
    base = wid * _RPW
    pltpu.sync_copy(perm_hbm.at[pl.ds(base, _RPW)], idx_v)
    pats = _patterns()
    rows = (rows0_v, rows1_v)
    gsems = (gsem0, gsem1)

    def gather_start(k, p):
        pltpu.async_copy(
            xt_hbm.at[idx_v.at[pl.ds(k * _G, _G)]], rows[p], gsems[p]
        )

    def gather_wait(k, p):
        pltpu.make_async_copy(
            xt_hbm.at[idx_v.at[pl.ds(k * _G, _G)]], rows[p], gsems[p]
        ).wait()

    def compute(k, p):
        buf = rows[p]

        def half(dt):
            @plsc.parallel_loop(0, 128, unroll=2)
            def _(u):
                s = 256 * dt + u
                for kk in range(8):
                    v = buf[128 * dt + u, pl.ds(16 * kk, 16)]
                    plsc.store_scatter(out_v, [pats[kk] + s], v)

        half(0)
        half(1)

    def writes_start(k):
        for b in range(B):
            start = b * _BSTR + (wid * 128 + 4 * k) * 128
            pltpu.async_copy(
                out_v.at[pl.ds(b * 512, 512)], out_hbm.at[pl.ds(start, 512)], wsem
            )

    def writes_wait(k):
        for b in range(B):
            start = b * _BSTR + (wid * 128 + 4 * k) * 128
            pltpu.make_async_copy(
                out_v.at[pl.ds(b * 512, 512)], out_hbm.at[pl.ds(start, 512)], wsem
            ).wait()

    gather_start(0, 0)

    def body(m, carry):
        k = 2 * m
        gather_start(k + 1, 1)
        gather_wait(k, 0)
        compute(k, 0)
        writes_start(k)

        @pl.when(m + 1 < _NCH // 2)
        def _():
            gather_start(k + 2, 0)

        gather_wait(k + 1, 1)
        writes_wait(k)
        compute(k + 1, 1)
        writes_start(k + 1)
        writes_wait(k + 1)
        return carry

    lax.fori_loop(0, _NCH // 2, body, 0)


def _raw_in_view(x):
    x4 = x.reshape(B, _TB, 128, C)
    x4 = with_layout_constraint(
        x4, Layout(major_to_minor=(0, 1, 3, 2), tiling=((2, 128),))
    )
    x5 = jnp.transpose(x4, (0, 1, 3, 2))
    x5 = with_layout_constraint(
        x5, Layout(major_to_minor=(0, 1, 2, 3), tiling=((2, 128),))
    )
    return x5.reshape(B * N * C)


def _raw_out_view(o):
    o5 = o.reshape(B, _TB, C, 128)
    o5 = with_layout_constraint(
        o5, Layout(major_to_minor=(0, 1, 2, 3), tiling=((2, 128),))
    )
    o4 = jnp.transpose(o5, (0, 1, 3, 2))
    o4 = with_layout_constraint(
        o4, Layout(major_to_minor=(0, 1, 3, 2), tiling=((2, 128),))
    )
    return o4.reshape(B, N, C)


def kernel(x, perm):
    xt = _transpose_kernel(_raw_in_view(x))
    return _raw_out_view(_gather_kernel(xt, perm))
